# baseline (speedup 1.0000x reference)
import jax
import jax.numpy as jnp
from jax import lax
from jax.experimental import pallas as pl
from jax.experimental.pallas import tpu as pltpu
from jax.experimental.pallas import tpu_sc as plsc

NUM_SCIENTISTS = 100000
NUM_PAPERS = 1000000
D = 64
GLOBAL_MEAN = 3.5
IMPLICIT_WEIGHT = 0.5
B = 4096
L = 50

NC, NS = 2, 16
NW = NC * NS
E = B // NW
C = 16
NCH = E // C
DV = D // 16

_LANE_IOTA = None


def _vgather(x, idx):
    dn = lax.GatherDimensionNumbers(
        offset_dims=(), collapsed_slice_dims=(0,), start_index_map=(0,))
    return lax.gather(x, idx[:, None], dn, (1,),
                      mode=lax.GatherScatterMode.PROMISE_IN_BOUNDS)


def _splat(x, i):
    return _vgather(x, jnp.full((16,), i, dtype=jnp.int32))


def _lanesum(t, lane):
    for k in (8, 4, 2, 1):
        t = t + _vgather(t, lane ^ jnp.int32(k))
    return t


def _sc_body(sids_hbm, pids_hbm, imp_hbm, lens_hbm, p_hbm, q_hbm,
             bs_hbm, bp_hbm, out_hbm,
             sids_v, pids_v, lens_v, imp_idx_v, imp_rows_v,
             p_rows_v, q_rows_v, bs_v, bp_v, out_v, sem):
    cid = lax.axis_index("c")
    scid = lax.axis_index("s")
    wid = scid * NC + cid

    pltpu.sync_copy(sids_hbm.at[wid], sids_v)
    pltpu.sync_copy(pids_hbm.at[wid], pids_v)
    pltpu.sync_copy(lens_hbm.at[wid], lens_v)
    pltpu.sync_copy(imp_hbm.at[wid], imp_idx_v)

    lane = lax.iota(jnp.int32, 16)

    def chunk(ch, carry):
        cps = []
        for i in range(C):
            cps.append(pltpu.async_copy(
                q_hbm.at[imp_idx_v.at[ch, i]], imp_rows_v.at[i], sem))
        cps.append(pltpu.async_copy(p_hbm.at[sids_v.at[ch]], p_rows_v, sem))
        cps.append(pltpu.async_copy(q_hbm.at[pids_v.at[ch]], q_rows_v, sem))
        cps.append(pltpu.async_copy(bs_hbm.at[sids_v.at[ch]], bs_v, sem))
        cps.append(pltpu.async_copy(bp_hbm.at[pids_v.at[ch]], bp_v, sem))
        for cp in cps:
            cp.wait()

        lens = lens_v[ch, :]
        lens_f = lens.astype(jnp.float32)
        h = 0.5 * lens_f
        yb = jnp.int32(0x5F3759DF) - (lax.bitcast_convert_type(
            lens_f, jnp.int32) >> 1)
        y = lax.bitcast_convert_type(yb, jnp.float32)
        for _ in range(3):
            y = y * (1.5 - h * y * y)
        sqrt_n = lens_f * y
        alpha = IMPLICIT_WEIGHT / (sqrt_n + 1e-9)

        def elem(i, out_vec):
            len_i = _splat(lens, i)
            a_i = _splat(alpha, i)
            zero = jnp.zeros((16,), jnp.float32)
            one = jnp.int32(1)
            zeroi = jnp.int32(0)
            acc = [zero, zero, zero, zero]
            for l in range(L):
                mf = jnp.minimum(jnp.maximum(len_i - jnp.int32(l), zeroi),
                                 one).astype(jnp.float32)
                for d in range(DV):
                    acc[d] = acc[d] + mf * imp_rows_v[i, l, pl.ds(d * 16, 16)]
            t = zero
            for d in range(DV):
                u = p_rows_v[i, pl.ds(d * 16, 16)] + a_i * acc[d]
                t = t + q_rows_v[i, pl.ds(d * 16, 16)] * u
            tot = _lanesum(t, lane)
            eq = jnp.minimum(jnp.maximum(one - jnp.abs(lane - i), zeroi),
                             one).astype(jnp.float32)
            return out_vec + eq * tot

        out_vec = lax.fori_loop(0, C, elem, jnp.zeros((16,), jnp.float32))
        out_v[ch, :] = GLOBAL_MEAN + bs_v[:] + bp_v[:] + out_vec
        return carry

    lax.fori_loop(0, NCH, chunk, jnp.int32(0))
    pltpu.sync_copy(out_v, out_hbm.at[wid])


def kernel(SIDs, PIDs, implicit_PIDs, implicit_lengths, P, Q,
           scientist_bias, paper_bias):
    mesh = plsc.VectorSubcoreMesh(core_axis_name="c", subcore_axis_name="s",
                                  num_cores=NC, num_subcores=NS)
    run = pl.kernel(
        _sc_body,
        out_type=jax.ShapeDtypeStruct((NW, NCH, C), jnp.float32),
        mesh=mesh,
        compiler_params=pltpu.CompilerParams(use_tc_tiling_on_sc=False),
        scratch_types=[
            pltpu.VMEM((NCH, C), jnp.int32),
            pltpu.VMEM((NCH, C), jnp.int32),
            pltpu.VMEM((NCH, C), jnp.int32),
            pltpu.VMEM((NCH, C, L), jnp.int32),
            pltpu.VMEM((C, L, D), jnp.float32),
            pltpu.VMEM((C, D), jnp.float32),
            pltpu.VMEM((C, D), jnp.float32),
            pltpu.VMEM((C,), jnp.float32),
            pltpu.VMEM((C,), jnp.float32),
            pltpu.VMEM((NCH, C), jnp.float32),
            pltpu.SemaphoreType.DMA,
        ],
    )
    out = run(
        SIDs.reshape(NW, NCH, C).astype(jnp.int32),
        PIDs.reshape(NW, NCH, C).astype(jnp.int32),
        implicit_PIDs.reshape(NW, NCH, C, L).astype(jnp.int32),
        implicit_lengths.reshape(NW, NCH, C).astype(jnp.int32),
        P,
        Q,
        scientist_bias.reshape(NUM_SCIENTISTS),
        paper_bias.reshape(NUM_PAPERS),
    )
    probe_out = _probe(Q.T, implicit_PIDs.reshape(NW, E * L).astype(jnp.int32))
    return out.reshape(B) + 0.0 * probe_out.sum()


def _probe_body(qt_hbm, imp_flat_hbm, out_hbm,
                spmem_q, idx_v, dst_v, out_small_v, sem):
    cid = lax.axis_index("c")
    scid = lax.axis_index("s")
    wid = scid * NC + cid

    pltpu.sync_copy(imp_flat_hbm.at[wid, 0], idx_v)

    @pl.when(scid == 0)
    def _stage():
        pltpu.sync_copy(qt_hbm.at[0], spmem_q)

    plsc.subcore_barrier()

    def probe(p, carry):
        pltpu.async_copy(spmem_q.at[idx_v], dst_v, sem).wait()
        return carry

    lax.fori_loop(0, 64, probe, jnp.int32(0))
    out_small_v[...] = dst_v[pl.ds(0, 16)]
    pltpu.sync_copy(out_small_v, out_hbm.at[wid])


def _probe(QT, imp_flat):
    mesh = plsc.VectorSubcoreMesh(core_axis_name="c", subcore_axis_name="s",
                                  num_cores=NC, num_subcores=NS)
    run = pl.kernel(
        _probe_body,
        out_type=jax.ShapeDtypeStruct((NW, 16), jnp.float32),
        mesh=mesh,
        compiler_params=pltpu.CompilerParams(use_tc_tiling_on_sc=False),
        scratch_types=[
            pltpu.VMEM_SHARED((NUM_PAPERS,), jnp.float32),
            pltpu.VMEM((E * L // 2,), jnp.int32),
            pltpu.VMEM((E * L // 2,), jnp.float32),
            pltpu.VMEM((16,), jnp.float32),
            pltpu.SemaphoreType.DMA,
        ],
    )
    return run(QT, imp_flat.reshape(NW, 2, E * L // 2))


# speedup vs baseline: 7.8135x; 7.8135x over previous
import jax
import jax.numpy as jnp
from jax import lax
from jax.experimental import pallas as pl
from jax.experimental.pallas import tpu as pltpu
from jax.experimental.pallas import tpu_sc as plsc

NUM_SCIENTISTS = 100000
NUM_PAPERS = 1000000
D = 64
GLOBAL_MEAN = 3.5
IMPLICIT_WEIGHT = 0.5
B = 4096
L = 50

NC, NS = 2, 16
NW = NC * NS
E = B // NW
C = 16
NCH = E // C
DV = D // 16


def _vgather(x, idx):
    dn = lax.GatherDimensionNumbers(
        offset_dims=(), collapsed_slice_dims=(0,), start_index_map=(0,))
    return lax.gather(x, idx[:, None], dn, (1,),
                      mode=lax.GatherScatterMode.PROMISE_IN_BOUNDS)


def _splat(x, i):
    return _vgather(x, jnp.full((16,), i, dtype=jnp.int32))


def _lanesum(t, lane):
    for k in (8, 4, 2, 1):
        t = t + _vgather(t, lane ^ jnp.int32(k))
    return t


def _sc_body(sids_hbm, pids_hbm, imp_hbm, lens_hbm, p_hbm, q_hbm,
             bs_hbm, bp_hbm, out_hbm,
             sids_v, pids_v, lens_v, impf_v,
             rows0_v, rows1_v, p0_v, p1_v, q0_v, q1_v,
             bs0_v, bs1_v, bp0_v, bp1_v, out_v, sem0, sem1):
    cid = lax.axis_index("c")
    scid = lax.axis_index("s")
    wid = scid * NC + cid

    pltpu.sync_copy(sids_hbm.at[wid], sids_v)
    pltpu.sync_copy(pids_hbm.at[wid], pids_v)
    pltpu.sync_copy(lens_hbm.at[wid], lens_v)
    pltpu.sync_copy(imp_hbm.at[wid], impf_v)

    lane = lax.iota(jnp.int32, 16)
    bufs = ((rows0_v, p0_v, q0_v, bs0_v, bp0_v, sem0),
            (rows1_v, p1_v, q1_v, bs1_v, bp1_v, sem1))

    def pairs(ch, k):
        rows, pb, qb, bsb, bpb, sem = bufs[k]
        return [(q_hbm.at[impf_v.at[ch]], rows),
                (p_hbm.at[sids_v.at[ch]], pb),
                (q_hbm.at[pids_v.at[ch]], qb),
                (bs_hbm.at[sids_v.at[ch]], bsb),
                (bp_hbm.at[pids_v.at[ch]], bpb)], sem

    def fire(ch, k):
        prs, sem = pairs(ch, k)
        for s, d in prs:
            pltpu.async_copy(s, d, sem)

    def drain(ch, k):
        prs, sem = pairs(ch, k)
        for s, d in prs:
            pltpu.make_async_copy(s, d, sem).wait()

    def compute(ch, k):
        rows_v, p_v, q_v, bs_v, bp_v, _ = bufs[k]
        lens = lens_v[ch, :]
        lens_f = lens.astype(jnp.float32)
        h = 0.5 * lens_f
        yb = jnp.int32(0x5F3759DF) - (lax.bitcast_convert_type(
            lens_f, jnp.int32) >> 1)
        y = lax.bitcast_convert_type(yb, jnp.float32)
        for _ in range(3):
            y = y * (1.5 - h * y * y)
        sqrt_n = lens_f * y
        alpha = IMPLICIT_WEIGHT / (sqrt_n + 1e-9)

        def elem(i, out_vec):
            len_i = _splat(lens, i)
            a_i = _splat(alpha, i)
            zero = jnp.zeros((16,), jnp.float32)
            one = jnp.int32(1)
            zeroi = jnp.int32(0)
            acc = [zero, zero, zero, zero]
            for l in range(L):
                mf = jnp.minimum(jnp.maximum(len_i - jnp.int32(l), zeroi),
                                 one).astype(jnp.float32)
                r = i * L + l
                for d in range(DV):
                    acc[d] = acc[d] + mf * rows_v[r, pl.ds(d * 16, 16)]
            t = zero
            for d in range(DV):
                u = p_v[i, pl.ds(d * 16, 16)] + a_i * acc[d]
                t = t + q_v[i, pl.ds(d * 16, 16)] * u
            tot = _lanesum(t, lane)
            eq = jnp.minimum(jnp.maximum(one - jnp.abs(lane - i), zeroi),
                             one).astype(jnp.float32)
            return out_vec + eq * tot

        out_vec = lax.fori_loop(0, C, elem, jnp.zeros((16,), jnp.float32))
        out_v[ch, :] = GLOBAL_MEAN + bs_v[:] + bp_v[:] + out_vec

    fire(jnp.int32(0), 0)

    def body(g, carry):
        ch0 = 2 * g
        ch1 = 2 * g + 1
        chn = jnp.minimum(2 * g + 2, NCH - 1)
        drain(ch0, 0)
        fire(ch1, 1)
        compute(ch0, 0)
        drain(ch1, 1)
        fire(chn, 0)
        compute(ch1, 1)
        return carry

    lax.fori_loop(0, NCH // 2, body, jnp.int32(0))
    drain(jnp.int32(NCH - 1), 0)
    pltpu.sync_copy(out_v, out_hbm.at[wid])


def kernel(SIDs, PIDs, implicit_PIDs, implicit_lengths, P, Q,
           scientist_bias, paper_bias):
    mesh = plsc.VectorSubcoreMesh(core_axis_name="c", subcore_axis_name="s",
                                  num_cores=NC, num_subcores=NS)
    run = pl.kernel(
        _sc_body,
        out_type=jax.ShapeDtypeStruct((NW, NCH, C), jnp.float32),
        mesh=mesh,
        compiler_params=pltpu.CompilerParams(use_tc_tiling_on_sc=False),
        scratch_types=[
            pltpu.VMEM((NCH, C), jnp.int32),
            pltpu.VMEM((NCH, C), jnp.int32),
            pltpu.VMEM((NCH, C), jnp.int32),
            pltpu.VMEM((NCH, C * L), jnp.int32),
            pltpu.VMEM((C * L, D), jnp.float32),
            pltpu.VMEM((C * L, D), jnp.float32),
            pltpu.VMEM((C, D), jnp.float32),
            pltpu.VMEM((C, D), jnp.float32),
            pltpu.VMEM((C, D), jnp.float32),
            pltpu.VMEM((C, D), jnp.float32),
            pltpu.VMEM((C,), jnp.float32),
            pltpu.VMEM((C,), jnp.float32),
            pltpu.VMEM((C,), jnp.float32),
            pltpu.VMEM((C,), jnp.float32),
            pltpu.VMEM((NCH, C), jnp.float32),
            pltpu.SemaphoreType.DMA,
            pltpu.SemaphoreType.DMA,
        ],
    )
    out = run(
        SIDs.reshape(NW, NCH, C).astype(jnp.int32),
        PIDs.reshape(NW, NCH, C).astype(jnp.int32),
        implicit_PIDs.reshape(NW, NCH, C * L).astype(jnp.int32),
        implicit_lengths.reshape(NW, NCH, C).astype(jnp.int32),
        P,
        Q,
        scientist_bias.reshape(NUM_SCIENTISTS),
        paper_bias.reshape(NUM_PAPERS),
    )
    return out.reshape(B)


# speedup vs baseline: 8.4809x; 1.0854x over previous
import jax
import jax.numpy as jnp
from jax import lax
from jax.experimental import pallas as pl
from jax.experimental.pallas import tpu as pltpu
from jax.experimental.pallas import tpu_sc as plsc

NUM_SCIENTISTS = 100000
NUM_PAPERS = 1000000
D = 64
GLOBAL_MEAN = 3.5
IMPLICIT_WEIGHT = 0.5
B = 4096
L = 50

NC, NS = 2, 16
NW = NC * NS
E = B // NW
C = 16
NCH = E // C
DV = D // 16
HC = C // 2
HROWS = HC * L
NG = NCH * 2


def _vgather(x, idx):
    dn = lax.GatherDimensionNumbers(
        offset_dims=(), collapsed_slice_dims=(0,), start_index_map=(0,))
    return lax.gather(x, idx[:, None], dn, (1,),
                      mode=lax.GatherScatterMode.PROMISE_IN_BOUNDS)


def _splat(x, i):
    return _vgather(x, jnp.full((16,), i, dtype=jnp.int32))


def _lanesum(t, lane):
    for k in (8, 4, 2, 1):
        t = t + _vgather(t, lane ^ jnp.int32(k))
    return t


def _sc_body(sids_hbm, pids_hbm, imp_hbm, lens_hbm, p_hbm, q_hbm,
             bs_hbm, bp_hbm, out_hbm,
             sids_v, pids_v, lens_v, impf_v,
             rows0_v, rows1_v, p0_v, p1_v, q0_v, q1_v,
             bs0_v, bs1_v, bp0_v, bp1_v, out_v, sem0, sem1):
    cid = lax.axis_index("c")
    scid = lax.axis_index("s")
    wid = scid * NC + cid

    pltpu.sync_copy(sids_hbm.at[wid], sids_v)
    pltpu.sync_copy(pids_hbm.at[wid], pids_v)
    pltpu.sync_copy(lens_hbm.at[wid], lens_v)
    pltpu.sync_copy(imp_hbm.at[wid], impf_v)

    lane = lax.iota(jnp.int32, 16)
    rbufs = (rows0_v, rows1_v)
    sems = (sem0, sem1)
    pqbufs = ((p0_v, q0_v, bs0_v, bp0_v), (p1_v, q1_v, bs1_v, bp1_v))

    def pairs(g, rk, pqk, even):
        ch = g // 2
        half = g % 2
        prs = [(q_hbm.at[impf_v.at[ch, pl.ds((g % 2) * HROWS, HROWS)]],
                rbufs[rk])]
        if even:
            pv, qv, bsv, bpv = pqbufs[pqk]
            prs += [(p_hbm.at[sids_v.at[ch]], pv),
                    (q_hbm.at[pids_v.at[ch]], qv),
                    (bs_hbm.at[sids_v.at[ch]], bsv),
                    (bp_hbm.at[pids_v.at[ch]], bpv)]
        return prs, sems[rk]

    def fire(g, rk, pqk, even):
        prs, sem = pairs(g, rk, pqk, even)
        for s, d in prs:
            pltpu.async_copy(s, d, sem)

    def drain(g, rk, pqk, even):
        prs, sem = pairs(g, rk, pqk, even)
        for s, d in prs:
            pltpu.make_async_copy(s, d, sem).wait()

    def compute(g, rk, pqk, half):
        ch = g // 2
        rows_v = rbufs[rk]
        p_v, q_v, bs_v, bp_v = pqbufs[pqk]
        lens = lens_v[ch, :]
        lens_f = lens.astype(jnp.float32)
        h = 0.5 * lens_f
        yb = jnp.int32(0x5F3759DF) - (lax.bitcast_convert_type(
            lens_f, jnp.int32) >> 1)
        y = lax.bitcast_convert_type(yb, jnp.float32)
        for _ in range(3):
            y = y * (1.5 - h * y * y)
        sqrt_n = lens_f * y
        alpha = IMPLICIT_WEIGHT / (sqrt_n + 1e-9)

        def elem(i, out_vec):
            len_i = _splat(lens, i)
            a_i = _splat(alpha, i)
            zero = jnp.zeros((16,), jnp.float32)
            one = jnp.int32(1)
            zeroi = jnp.int32(0)
            loc = i - half * HC
            acc = [zero, zero, zero, zero]
            for l in range(L):
                mf = jnp.minimum(jnp.maximum(len_i - jnp.int32(l), zeroi),
                                 one).astype(jnp.float32)
                r = loc * L + l
                for d in range(DV):
                    acc[d] = acc[d] + mf * rows_v[r, pl.ds(d * 16, 16)]
            t = zero
            for d in range(DV):
                u = p_v[i, pl.ds(d * 16, 16)] + a_i * acc[d]
                t = t + q_v[i, pl.ds(d * 16, 16)] * u
            tot = _lanesum(t, lane)
            eq = jnp.minimum(jnp.maximum(one - jnp.abs(lane - i), zeroi),
                             one).astype(jnp.float32)
            return out_vec + eq * tot

        out_vec = lax.fori_loop(half * HC, half * HC + HC, elem,
                                jnp.zeros((16,), jnp.float32))
        if half == 0:
            out_v[ch, :] = GLOBAL_MEAN + bs_v[:] + bp_v[:] + out_vec
        else:
            out_v[ch, :] = out_v[ch, :] + out_vec

    fire(jnp.int32(0), 0, 0, True)

    def body(s, carry):
        for j in range(4):
            g = 4 * s + j
            rk = j % 2
            pqk = j // 2
            half = j % 2
            gn = jnp.minimum(g + 1, NG - 1)
            rkn = (j + 1) % 2
            pqkn = ((j + 1) // 2) % 2
            evenn = ((j + 1) % 2 == 0)
            drain(g, rk, pqk, half == 0)
            fire(gn, rkn, pqkn, evenn)
            compute(g, rk, pqk, half)
        return carry

    lax.fori_loop(0, NG // 4, body, jnp.int32(0))
    drain(jnp.int32(NG - 1), 0, 0, True)
    pltpu.sync_copy(out_v, out_hbm.at[wid])


def kernel(SIDs, PIDs, implicit_PIDs, implicit_lengths, P, Q,
           scientist_bias, paper_bias):
    mesh = plsc.VectorSubcoreMesh(core_axis_name="c", subcore_axis_name="s",
                                  num_cores=NC, num_subcores=NS)
    run = pl.kernel(
        _sc_body,
        out_type=jax.ShapeDtypeStruct((NW, NCH, C), jnp.float32),
        mesh=mesh,
        compiler_params=pltpu.CompilerParams(use_tc_tiling_on_sc=False),
        scratch_types=[
            pltpu.VMEM((NCH, C), jnp.int32),
            pltpu.VMEM((NCH, C), jnp.int32),
            pltpu.VMEM((NCH, C), jnp.int32),
            pltpu.VMEM((NCH, C * L), jnp.int32),
            pltpu.VMEM((HROWS, 128), jnp.float32),
            pltpu.VMEM((HROWS, 128), jnp.float32),
            pltpu.VMEM((C, 128), jnp.float32),
            pltpu.VMEM((C, 128), jnp.float32),
            pltpu.VMEM((C, 128), jnp.float32),
            pltpu.VMEM((C, 128), jnp.float32),
            pltpu.VMEM((C,), jnp.float32),
            pltpu.VMEM((C,), jnp.float32),
            pltpu.VMEM((C,), jnp.float32),
            pltpu.VMEM((C,), jnp.float32),
            pltpu.VMEM((NCH, C), jnp.float32),
            pltpu.SemaphoreType.DMA,
            pltpu.SemaphoreType.DMA,
        ],
    )
    Qp = jnp.pad(Q, ((0, 0), (0, 128 - D)))
    Pp = jnp.pad(P, ((0, 0), (0, 128 - D)))
    out = run(
        SIDs.reshape(NW, NCH, C).astype(jnp.int32),
        PIDs.reshape(NW, NCH, C).astype(jnp.int32),
        implicit_PIDs.reshape(NW, NCH, C * L).astype(jnp.int32),
        implicit_lengths.reshape(NW, NCH, C).astype(jnp.int32),
        Pp,
        Qp,
        scientist_bias.reshape(NUM_SCIENTISTS),
        paper_bias.reshape(NUM_PAPERS),
    )
    return out.reshape(B)
